# baseline (speedup 1.0000x reference)
import functools

import jax
import jax.numpy as jnp
from jax import lax
from jax.experimental import pallas as pl
from jax.experimental.pallas import tpu as pltpu
from jax.experimental.pallas import tpu_sc as plsc

INP_DIM = 64
TAG_DIM = 16
OUT_DIM = INP_DIM + 2 * TAG_DIM
CHUNK = 128
K = 5


@functools.cache
def _build(n_rows: int):
    info = plsc.get_sparse_core_info()
    nw = info.num_cores * info.num_subcores
    assert n_rows % (nw * CHUNK * K) == 0
    per_w = n_rows // nw
    n_chunks = per_w // CHUNK
    n_blocks = n_chunks // K

    mesh = plsc.VectorSubcoreMesh(core_axis_name="c", subcore_axis_name="s")

    scratch = (
        [pltpu.VMEM((n_chunks, CHUNK), jnp.int32)] * 3
        + [pltpu.VMEM((CHUNK, INP_DIM), jnp.float32) for _ in range(K)]
        + [pltpu.VMEM((CHUNK, TAG_DIM), jnp.float32) for _ in range(2 * K)]
        + [pltpu.SemaphoreType.DMA for _ in range(2 * K)]
    )

    @functools.partial(
        pl.kernel,
        mesh=mesh,
        out_type=jax.ShapeDtypeStruct((n_rows, OUT_DIM), jnp.float32),
        scratch_types=scratch,
        compiler_params=pltpu.CompilerParams(use_tc_tiling_on_sc=False),
    )
    def k(widx_hbm, pidx_hbm, nidx_hbm, wtab_hbm, ptab_hbm, ntab_hbm,
          out_hbm, widx_v, pidx_v, nidx_v, *bufs):
        wrows = bufs[:K]
        prows = bufs[K:2 * K]
        nrows = bufs[2 * K:3 * K]
        gsem = bufs[3 * K:4 * K]
        osem = bufs[4 * K:5 * K]

        wid = lax.axis_index("s") * info.num_cores + lax.axis_index("c")
        pltpu.sync_copy(widx_hbm.at[wid], widx_v)
        pltpu.sync_copy(pidx_hbm.at[wid], pidx_v)
        pltpu.sync_copy(nidx_hbm.at[wid], nidx_v)

        def body(g, carry):
            gathers = []
            for b in range(K):
                c = g * K + b
                gathers.append((
                    pltpu.async_copy(
                        wtab_hbm.at[widx_v.at[c]], wrows[b], gsem[b]),
                    pltpu.async_copy(
                        ptab_hbm.at[pidx_v.at[c]], prows[b], gsem[b]),
                    pltpu.async_copy(
                        ntab_hbm.at[nidx_v.at[c]], nrows[b], gsem[b]),
                ))
            writes = []
            for b in range(K):
                for d in gathers[b]:
                    d.wait()
                base = wid * per_w + (g * K + b) * CHUNK
                writes.append((
                    pltpu.async_copy(
                        wrows[b],
                        out_hbm.at[pl.ds(base, CHUNK), pl.ds(0, INP_DIM)],
                        osem[b]),
                    pltpu.async_copy(
                        prows[b],
                        out_hbm.at[pl.ds(base, CHUNK),
                                   pl.ds(INP_DIM, TAG_DIM)],
                        osem[b]),
                    pltpu.async_copy(
                        nrows[b],
                        out_hbm.at[pl.ds(base, CHUNK),
                                   pl.ds(INP_DIM + TAG_DIM, TAG_DIM)],
                        osem[b]),
                ))
            for b in range(K):
                for d in writes[b]:
                    d.wait()
            return carry

        lax.fori_loop(0, n_blocks, body, 0, unroll=False)

    def run(widx, pidx, nidx, wtab, ptab, ntab):
        shp = (nw, n_chunks, CHUNK)
        return k(widx.reshape(shp), pidx.reshape(shp), nidx.reshape(shp),
                 wtab, ptab, ntab)

    return run


def kernel(seq_word, seq_pos, seq_ner, word_table, pos_table, ner_table):
    s, b = seq_word.shape
    n = s * b
    run = _build(n)
    out = run(
        seq_word.reshape(n).astype(jnp.int32),
        seq_pos.reshape(n).astype(jnp.int32),
        seq_ner.reshape(n).astype(jnp.int32),
        word_table, pos_table, ner_table)
    return out.reshape(s, b, OUT_DIM)


# speedup vs baseline: 1.0876x; 1.0876x over previous
import functools

import jax
import jax.numpy as jnp
from jax import lax
from jax.experimental import pallas as pl
from jax.experimental.pallas import tpu as pltpu
from jax.experimental.pallas import tpu_sc as plsc

INP_DIM = 64
TAG_DIM = 16
OUT_DIM = INP_DIM + 2 * TAG_DIM
PAD_DIM = 128
CHUNK = 128
K = 5
SB = 8
BB = 4


@functools.cache
def _build(s_len: int, b_len: int):
    info = plsc.get_sparse_core_info()
    nw = info.num_cores * info.num_subcores
    assert nw == SB * BB
    s_blk = s_len // SB
    b_blk = b_len // BB
    per_w = s_blk * b_blk
    n_chunks = per_w // CHUNK
    n_blocks = n_chunks // K
    halves = b_blk // CHUNK
    n_rows = s_len * b_len

    mesh = plsc.VectorSubcoreMesh(core_axis_name="c", subcore_axis_name="s")

    scratch = (
        [pltpu.VMEM((s_blk, b_blk), jnp.int32)] * 3
        + [pltpu.VMEM((CHUNK, INP_DIM), jnp.float32) for _ in range(K)]
        + [pltpu.VMEM((CHUNK, TAG_DIM), jnp.float32) for _ in range(2 * K)]
        + [pltpu.SemaphoreType.DMA for _ in range(2 * K)]
    )

    @functools.partial(
        pl.kernel,
        mesh=mesh,
        out_type=jax.ShapeDtypeStruct((n_rows, PAD_DIM), jnp.float32),
        scratch_types=scratch,
        compiler_params=pltpu.CompilerParams(use_tc_tiling_on_sc=False),
    )
    def k(widx_hbm, pidx_hbm, nidx_hbm, wtab_hbm, ptab_hbm, ntab_hbm,
          out_hbm, widx_v, pidx_v, nidx_v, *bufs):
        wrows = bufs[:K]
        prows = bufs[K:2 * K]
        nrows = bufs[2 * K:3 * K]
        gsem = bufs[3 * K:4 * K]
        osem = bufs[4 * K:5 * K]

        wid = lax.axis_index("s") * info.num_cores + lax.axis_index("c")
        si = (wid // BB) * s_blk
        bj = (wid % BB) * b_blk
        blk = (pl.ds(si, s_blk), pl.ds(bj, b_blk))
        pltpu.sync_copy(widx_hbm.at[blk[0], blk[1]], widx_v)
        pltpu.sync_copy(pidx_hbm.at[blk[0], blk[1]], pidx_v)
        pltpu.sync_copy(nidx_hbm.at[blk[0], blk[1]], nidx_v)

        def body(g, carry):
            gathers = []
            for b in range(K):
                c = g * K + b
                r = c // halves
                off = (c % halves) * CHUNK
                gathers.append((
                    pltpu.async_copy(
                        wtab_hbm.at[widx_v.at[r, pl.ds(off, CHUNK)]],
                        wrows[b], gsem[b]),
                    pltpu.async_copy(
                        ptab_hbm.at[pidx_v.at[r, pl.ds(off, CHUNK)]],
                        prows[b], gsem[b]),
                    pltpu.async_copy(
                        ntab_hbm.at[nidx_v.at[r, pl.ds(off, CHUNK)]],
                        nrows[b], gsem[b]),
                ))
            writes = []
            for b in range(K):
                for d in gathers[b]:
                    d.wait()
                c = g * K + b
                r = c // halves
                off = (c % halves) * CHUNK
                base = (si + r) * b_len + bj + off
                writes.append((
                    pltpu.async_copy(
                        wrows[b],
                        out_hbm.at[pl.ds(base, CHUNK), pl.ds(0, INP_DIM)],
                        osem[b]),
                    pltpu.async_copy(
                        prows[b],
                        out_hbm.at[pl.ds(base, CHUNK),
                                   pl.ds(INP_DIM, TAG_DIM)],
                        osem[b]),
                    pltpu.async_copy(
                        nrows[b],
                        out_hbm.at[pl.ds(base, CHUNK),
                                   pl.ds(INP_DIM + TAG_DIM, TAG_DIM)],
                        osem[b]),
                ))
            for b in range(K):
                for d in writes[b]:
                    d.wait()
            return carry

        lax.fori_loop(0, n_blocks, body, 0, unroll=False)

    return k


def kernel(seq_word, seq_pos, seq_ner, word_table, pos_table, ner_table):
    s, b = seq_word.shape
    run = _build(s, b)
    out = run(
        seq_word.astype(jnp.int32), seq_pos.astype(jnp.int32),
        seq_ner.astype(jnp.int32), word_table, pos_table, ner_table)
    return out.reshape(s, b, PAD_DIM)[:, :, :OUT_DIM]
